# baseline (speedup 1.0000x reference)
import functools

import jax
import jax.numpy as jnp
from jax.experimental import pallas as pl
from jax.experimental.pallas import tpu as pltpu


def _feature_kernel(v_ref, w_ref, xh_ref, xl_ref):
    x = jnp.dot(v_ref[...], w_ref[...], preferred_element_type=jnp.float32)
    hi = x.astype(jnp.bfloat16)
    xh_ref[...] = hi
    xl_ref[...] = (x - hi.astype(jnp.float32)).astype(jnp.bfloat16)


def _spmm_kernel(adj_ref, xh_ref, xl_ref, bias_ref, out_ref, *, bn, out_f):
    j = pl.program_id(1)

    @pl.when(j == 0)
    def _init():
        out_ref[...] = jnp.broadcast_to(bias_ref[...], out_ref.shape)

    adj = adj_ref[...]
    xh = xh_ref[pl.ds(j * bn, bn), :]
    xl = xl_ref[pl.ds(j * bn, bn), :]
    a1 = (adj == 1).astype(jnp.bfloat16)
    a2 = (adj == 2).astype(jnp.bfloat16)
    a3 = (adj == 3).astype(jnp.bfloat16)
    acc = jnp.dot(a1, xh[:, :out_f], preferred_element_type=jnp.float32)
    acc += jnp.dot(a2, xh[:, out_f:2 * out_f],
                   preferred_element_type=jnp.float32)
    acc += jnp.dot(a3, xh[:, 2 * out_f:],
                   preferred_element_type=jnp.float32)
    acc += jnp.dot(a1, xl[:, :out_f], preferred_element_type=jnp.float32)
    acc += jnp.dot(a2, xl[:, out_f:2 * out_f],
                   preferred_element_type=jnp.float32)
    acc += jnp.dot(a3, xl[:, 2 * out_f:],
                   preferred_element_type=jnp.float32)
    out_ref[...] += acc


def kernel(V, adj, w1, w2, w3, bias):
    n, in_f = V.shape
    out_f = w1.shape[1]
    w = jnp.concatenate([w1, w2, w3], axis=1)

    bm_x = 1024
    xh, xl = pl.pallas_call(
        _feature_kernel,
        grid=(n // bm_x,),
        in_specs=[
            pl.BlockSpec((bm_x, in_f), lambda i: (i, 0)),
            pl.BlockSpec((in_f, 3 * out_f), lambda i: (0, 0)),
        ],
        out_specs=[
            pl.BlockSpec((bm_x, 3 * out_f), lambda i: (i, 0)),
            pl.BlockSpec((bm_x, 3 * out_f), lambda i: (i, 0)),
        ],
        out_shape=[
            jax.ShapeDtypeStruct((n, 3 * out_f), jnp.bfloat16),
            jax.ShapeDtypeStruct((n, 3 * out_f), jnp.bfloat16),
        ],
    )(V, w)

    bm, bn = 1024, 1024
    body = functools.partial(_spmm_kernel, bn=bn, out_f=out_f)
    out = pl.pallas_call(
        body,
        grid=(n // bm, n // bn),
        in_specs=[
            pl.BlockSpec((bm, bn), lambda i, j: (i, j)),
            pl.BlockSpec((n, 3 * out_f), lambda i, j: (0, 0)),
            pl.BlockSpec((n, 3 * out_f), lambda i, j: (0, 0)),
            pl.BlockSpec((1, out_f), lambda i, j: (0, 0)),
        ],
        out_specs=pl.BlockSpec((bm, out_f), lambda i, j: (i, 0)),
        out_shape=jax.ShapeDtypeStruct((n, out_f), jnp.float32),
        compiler_params=pltpu.CompilerParams(
            dimension_semantics=("parallel", "arbitrary"),
        ),
    )(adj, xh, xl, bias.reshape(1, out_f))
    return out


# speedup vs baseline: 1.6543x; 1.6543x over previous
import functools

import jax
import jax.numpy as jnp
from jax.experimental import pallas as pl
from jax.experimental.pallas import tpu as pltpu


def _feature_kernel(v_ref, w_ref, x_ref):
    x_ref[...] = jnp.dot(v_ref[...], w_ref[...],
                         preferred_element_type=jnp.float32)


def _spmm_kernel(adj_ref, x_ref, bias_ref, out_ref, *, bn, out_f):
    j = pl.program_id(1)

    @pl.when(j == 0)
    def _init():
        out_ref[...] = jnp.broadcast_to(bias_ref[...], out_ref.shape)

    adj = adj_ref[...]
    xs = x_ref[pl.ds(j * bn, bn), :]
    a1 = (adj == 1).astype(jnp.float32)
    a2 = (adj == 2).astype(jnp.float32)
    a3 = (adj == 3).astype(jnp.float32)
    acc = jnp.dot(a1, xs[:, :out_f], preferred_element_type=jnp.float32)
    acc += jnp.dot(a2, xs[:, out_f:2 * out_f],
                   preferred_element_type=jnp.float32)
    acc += jnp.dot(a3, xs[:, 2 * out_f:],
                   preferred_element_type=jnp.float32)
    out_ref[...] += acc


def kernel(V, adj, w1, w2, w3, bias):
    n, in_f = V.shape
    out_f = w1.shape[1]
    w = jnp.concatenate([w1, w2, w3], axis=1)

    bm_x = 1024
    x = pl.pallas_call(
        _feature_kernel,
        grid=(n // bm_x,),
        in_specs=[
            pl.BlockSpec((bm_x, in_f), lambda i: (i, 0)),
            pl.BlockSpec((in_f, 3 * out_f), lambda i: (0, 0)),
        ],
        out_specs=pl.BlockSpec((bm_x, 3 * out_f), lambda i: (i, 0)),
        out_shape=jax.ShapeDtypeStruct((n, 3 * out_f), jnp.float32),
    )(V, w)

    bm, bn = 1024, 1024
    body = functools.partial(_spmm_kernel, bn=bn, out_f=out_f)
    out = pl.pallas_call(
        body,
        grid=(n // bm, n // bn),
        in_specs=[
            pl.BlockSpec((bm, bn), lambda i, j: (i, j)),
            pl.BlockSpec((n, 3 * out_f), lambda i, j: (0, 0)),
            pl.BlockSpec((1, out_f), lambda i, j: (0, 0)),
        ],
        out_specs=pl.BlockSpec((bm, out_f), lambda i, j: (i, 0)),
        out_shape=jax.ShapeDtypeStruct((n, out_f), jnp.float32),
        compiler_params=pltpu.CompilerParams(
            dimension_semantics=("parallel", "arbitrary"),
        ),
    )(adj, x, bias.reshape(1, out_f))
    return out


# speedup vs baseline: 1.8660x; 1.1280x over previous
import functools

import jax
import jax.numpy as jnp
from jax.experimental import pallas as pl
from jax.experimental.pallas import tpu as pltpu


def _feature_kernel(v_ref, w_ref, x_ref):
    x_ref[...] = jnp.dot(v_ref[...], w_ref[...],
                         preferred_element_type=jnp.float32)


def _spmm_kernel(adj_ref, x_ref, bias_ref, out_ref, *, out_f):
    adj = adj_ref[...]
    xs = x_ref[...]
    a1 = (adj == 1).astype(jnp.float32)
    a2 = (adj == 2).astype(jnp.float32)
    a3 = (adj == 3).astype(jnp.float32)
    acc = jnp.dot(a1, xs[:, :out_f], preferred_element_type=jnp.float32)
    acc += jnp.dot(a2, xs[:, out_f:2 * out_f],
                   preferred_element_type=jnp.float32)
    acc += jnp.dot(a3, xs[:, 2 * out_f:],
                   preferred_element_type=jnp.float32)
    out_ref[...] = acc + bias_ref[...]


def kernel(V, adj, w1, w2, w3, bias):
    n, in_f = V.shape
    out_f = w1.shape[1]
    w = jnp.concatenate([w1, w2, w3], axis=1)

    bm_x = 1024
    x = pl.pallas_call(
        _feature_kernel,
        grid=(n // bm_x,),
        in_specs=[
            pl.BlockSpec((bm_x, in_f), lambda i: (i, 0)),
            pl.BlockSpec((in_f, 3 * out_f), lambda i: (0, 0)),
        ],
        out_specs=pl.BlockSpec((bm_x, 3 * out_f), lambda i: (i, 0)),
        out_shape=jax.ShapeDtypeStruct((n, 3 * out_f), jnp.float32),
    )(V, w)

    bm = 256
    body = functools.partial(_spmm_kernel, out_f=out_f)
    out = pl.pallas_call(
        body,
        grid=(n // bm,),
        in_specs=[
            pl.BlockSpec((bm, n), lambda i: (i, 0)),
            pl.BlockSpec((n, 3 * out_f), lambda i: (0, 0)),
            pl.BlockSpec((1, out_f), lambda i: (0, 0)),
        ],
        out_specs=pl.BlockSpec((bm, out_f), lambda i: (i, 0)),
        out_shape=jax.ShapeDtypeStruct((n, out_f), jnp.float32),
        compiler_params=pltpu.CompilerParams(
            dimension_semantics=("arbitrary",),
        ),
    )(adj, x, bias.reshape(1, out_f))
    return out
